# baseline (speedup 1.0000x reference)
import functools

import jax
import jax.numpy as jnp
from jax import lax
from jax.experimental import pallas as pl
from jax.experimental.pallas import tpu as pltpu
from jax.experimental.pallas import tpu_sc as plsc

_NUM_CORES = 2
_NUM_SUBCORES = 16
_NW = _NUM_CORES * _NUM_SUBCORES
_IDX_CHUNK = 128


@functools.lru_cache(maxsize=None)
def _make_sc_gather(B, D):
    bpw = B // _NW
    nchunks = bpw // _IDX_CHUNK
    mesh = plsc.VectorSubcoreMesh(core_axis_name="c", subcore_axis_name="s")

    @functools.partial(
        pl.kernel,
        mesh=mesh,
        out_type=[jax.ShapeDtypeStruct((B, D), jnp.float32)] * 3,
        scratch_types=[
            pltpu.VMEM((nchunks, _IDX_CHUNK), jnp.int32),
            pltpu.VMEM((nchunks, _IDX_CHUNK), jnp.int32),
            pltpu.VMEM((nchunks, _IDX_CHUNK), jnp.int32),
            pltpu.VMEM((bpw, D), jnp.float32),
            pltpu.VMEM((bpw, D), jnp.float32),
            pltpu.VMEM((bpw, D), jnp.float32),
            pltpu.SemaphoreType.DMA,
        ],
        compiler_params=pltpu.CompilerParams(use_tc_tiling_on_sc=False),
    )
    def sc_gather(x_hbm, rel_hbm, hidx_hbm, ridx_hbm, tidx_hbm,
                  h_out, r_out, t_out,
                  hidx_v, ridx_v, tidx_v, hrows, rrows, trows, sem):
        wid = lax.axis_index("s") * _NUM_CORES + lax.axis_index("c")
        cbase = wid * nchunks
        pltpu.sync_copy(hidx_hbm.at[pl.ds(cbase, nchunks)], hidx_v)
        pltpu.sync_copy(ridx_hbm.at[pl.ds(cbase, nchunks)], ridx_v)
        pltpu.sync_copy(tidx_hbm.at[pl.ds(cbase, nchunks)], tidx_v)
        cps = []
        for j in range(nchunks):
            dst = pl.ds(j * _IDX_CHUNK, _IDX_CHUNK)
            cps.append(pltpu.async_copy(x_hbm.at[hidx_v.at[j]], hrows.at[dst], sem))
            cps.append(pltpu.async_copy(x_hbm.at[tidx_v.at[j]], trows.at[dst], sem))
            cps.append(pltpu.async_copy(rel_hbm.at[ridx_v.at[j]], rrows.at[dst], sem))
        for c in cps:
            c.wait()
        rbase = wid * bpw
        pltpu.sync_copy(hrows, h_out.at[pl.ds(rbase, bpw)])
        pltpu.sync_copy(rrows, r_out.at[pl.ds(rbase, bpw)])
        pltpu.sync_copy(trows, t_out.at[pl.ds(rbase, bpw)])

    return sc_gather


def _tc_score_body(h_ref, r_ref, t_ref, w_ref, cb_ref, g_ref, s_ref, l2_ref,
                   *, out_ch, d, l2_scale):
    i = pl.program_id(0)
    h = h_ref[...]
    r = r_ref[...]
    t = t_ref[...]
    acc = None
    for o in range(out_ch):
        pre = h * w_ref[o, 0] + r * w_ref[o, 1] + t * w_ref[o, 2] + cb_ref[o]
        z = jnp.maximum(pre, 0.0)
        term = z * g_ref[pl.ds(o, 1), :]
        acc = term if acc is None else acc + term
    s0 = jnp.sum(acc[:, :d], axis=1, keepdims=True)
    s1 = jnp.sum(acc[:, d:], axis=1, keepdims=True)
    s_ref[...] = jnp.concatenate([s0, s1], axis=1)
    part = (jnp.sum(h * h) + jnp.sum(r * r) + jnp.sum(t * t)) * l2_scale

    @pl.when(i == 0)
    def _():
        l2_ref[0, 0] = part

    @pl.when(i > 0)
    def _():
        l2_ref[0, 0] = l2_ref[0, 0] + part


def _tc_score(h2, r2, t2, w, cb, g2, *, out_ch, d, interpret=False):
    B2, L = h2.shape
    tile = 512
    nsteps = B2 // tile
    body = functools.partial(
        _tc_score_body, out_ch=out_ch, d=d, l2_scale=1.0 / (3.0 * B2 * L))
    return pl.pallas_call(
        body,
        grid=(nsteps,),
        in_specs=[
            pl.BlockSpec((tile, L), lambda i: (i, 0)),
            pl.BlockSpec((tile, L), lambda i: (i, 0)),
            pl.BlockSpec((tile, L), lambda i: (i, 0)),
            pl.BlockSpec(memory_space=pltpu.SMEM),
            pl.BlockSpec(memory_space=pltpu.SMEM),
            pl.BlockSpec((out_ch, L), lambda i: (0, 0)),
        ],
        out_specs=[
            pl.BlockSpec((tile, 2), lambda i: (i, 0)),
            pl.BlockSpec(memory_space=pltpu.SMEM),
        ],
        out_shape=[
            jax.ShapeDtypeStruct((B2, 2), jnp.float32),
            jax.ShapeDtypeStruct((1, 1), jnp.float32),
        ],
        compiler_params=pltpu.CompilerParams(
            dimension_semantics=("arbitrary",)),
        interpret=interpret,
    )(h2, r2, t2, w, cb, g2)


def kernel(x, rel_emb, head_index, rel_type, tail_index, conv_w, conv_b, fc_w):
    B = head_index.shape[0]
    D = x.shape[1]
    out_ch = conv_w.shape[0]

    hidx = head_index.astype(jnp.int32).reshape(B // _IDX_CHUNK, _IDX_CHUNK)
    ridx = rel_type.astype(jnp.int32).reshape(B // _IDX_CHUNK, _IDX_CHUNK)
    tidx = tail_index.astype(jnp.int32).reshape(B // _IDX_CHUNK, _IDX_CHUNK)

    sc_gather = _make_sc_gather(B, D)
    h, r, t = sc_gather(x, rel_emb, hidx, ridx, tidx)

    h2 = h.reshape(B // 2, 2 * D)
    r2 = r.reshape(B // 2, 2 * D)
    t2 = t.reshape(B // 2, 2 * D)
    w = conv_w.reshape(out_ch, 3)
    g = fc_w.reshape(out_ch, D)
    g2 = jnp.concatenate([g, g], axis=1)

    score2, l2s = _tc_score(h2, r2, t2, w, conv_b, g2, out_ch=out_ch, d=D)
    return score2.reshape(B), l2s[0, 0]


# speedup vs baseline: 1.3775x; 1.3775x over previous
import functools

import jax
import jax.numpy as jnp
from jax import lax
from jax.experimental import pallas as pl
from jax.experimental.pallas import tpu as pltpu
from jax.experimental.pallas import tpu_sc as plsc

_NUM_CORES = 2
_NUM_SUBCORES = 16
_NW = _NUM_CORES * _NUM_SUBCORES


@functools.lru_cache(maxsize=None)
def _make_sc_gather(B, D):
    bpw = B // _NW
    mesh = plsc.VectorSubcoreMesh(core_axis_name="c", subcore_axis_name="s")

    @functools.partial(
        pl.kernel,
        mesh=mesh,
        out_type=[jax.ShapeDtypeStruct((B, D), jnp.float32)] * 2,
        scratch_types=[
            pltpu.VMEM((bpw,), jnp.int32),
            pltpu.VMEM((bpw,), jnp.int32),
            pltpu.VMEM((bpw // 2, D), jnp.float32),
            pltpu.VMEM((bpw // 2, D), jnp.float32),
            pltpu.SemaphoreType.DMA,
        ],
        compiler_params=pltpu.CompilerParams(needs_layout_passes=False),
    )
    def sc_gather(x_hbm, hidx_hbm, tidx_hbm,
                  h_out, t_out,
                  hiv, tiv, hrows, trows, sem):
        wid = lax.axis_index("s") * _NUM_CORES + lax.axis_index("c")
        base = wid * bpw
        pltpu.sync_copy(hidx_hbm.at[pl.ds(base, bpw)], hiv)
        pltpu.sync_copy(tidx_hbm.at[pl.ds(base, bpw)], tiv)
        lanes = lax.broadcasted_iota(jnp.int32, (16,), 0)
        half = bpw // 2

        for p in range(2):
            def group(g, _, p=p):
                hvec = hiv[pl.ds(p * half + g * 16, 16)]
                tvec = tiv[pl.ds(p * half + g * 16, 16)]
                cps = []
                for i in range(16):
                    hidx = jnp.sum(jnp.where(lanes == i, hvec, 0))
                    tidx = jnp.sum(jnp.where(lanes == i, tvec, 0))
                    dst = pl.ds(g * 16 + i, 1)
                    cps.append(pltpu.async_copy(
                        x_hbm.at[pl.ds(hidx, 1)], hrows.at[dst], sem))
                    cps.append(pltpu.async_copy(
                        x_hbm.at[pl.ds(tidx, 1)], trows.at[dst], sem))
                for cp in cps:
                    cp.wait()
                return 0

            lax.fori_loop(0, half // 16, group, 0)
            dst = pl.ds(base + p * half, half)
            pltpu.sync_copy(hrows, h_out.at[dst])
            pltpu.sync_copy(trows, t_out.at[dst])

    return sc_gather


def _tc_score_body(h_ref, t_ref, ri_ref, rel_ref, w_ref, cb_ref, g_ref,
                   s_ref, l2_ref, *, out_ch, n_rel, l2_scale):
    i = pl.program_id(0)
    h = h_ref[...]
    t = t_ref[...]
    rows = h_ref.shape[0]
    onehot = (lax.broadcasted_iota(jnp.int32, (rows, n_rel), 1)
              == ri_ref[...]).astype(jnp.float32)
    r = jnp.dot(onehot, rel_ref[...], preferred_element_type=jnp.float32)
    acc = None
    for o in range(out_ch):
        pre = h * w_ref[o, 0] + r * w_ref[o, 1] + t * w_ref[o, 2] + cb_ref[o]
        z = jnp.maximum(pre, 0.0)
        term = z * g_ref[pl.ds(o, 1), :]
        acc = term if acc is None else acc + term
    s_ref[...] = jnp.sum(acc, axis=1, keepdims=True)
    part = (jnp.sum(h * h) + jnp.sum(t * t) + jnp.sum(r * r)) * l2_scale

    @pl.when(i == 0)
    def _():
        l2_ref[0, 0] = part

    @pl.when(i > 0)
    def _():
        l2_ref[0, 0] = l2_ref[0, 0] + part


def _tc_score(h, t, ri, rel, w, cb, g, *, out_ch, d, interpret=False):
    B = h.shape[0]
    n_rel = rel.shape[0]
    tile = 512
    nsteps = B // tile
    body = functools.partial(
        _tc_score_body, out_ch=out_ch, n_rel=n_rel,
        l2_scale=1.0 / (3.0 * B * d))
    return pl.pallas_call(
        body,
        grid=(nsteps,),
        in_specs=[
            pl.BlockSpec((tile, d), lambda i: (i, 0)),
            pl.BlockSpec((tile, d), lambda i: (i, 0)),
            pl.BlockSpec((tile, 1), lambda i: (i, 0)),
            pl.BlockSpec((n_rel, d), lambda i: (0, 0)),
            pl.BlockSpec(memory_space=pltpu.SMEM),
            pl.BlockSpec(memory_space=pltpu.SMEM),
            pl.BlockSpec((out_ch, d), lambda i: (0, 0)),
        ],
        out_specs=[
            pl.BlockSpec((tile, 1), lambda i: (i, 0)),
            pl.BlockSpec(memory_space=pltpu.SMEM),
        ],
        out_shape=[
            jax.ShapeDtypeStruct((B, 1), jnp.float32),
            jax.ShapeDtypeStruct((1, 1), jnp.float32),
        ],
        compiler_params=pltpu.CompilerParams(
            dimension_semantics=("arbitrary",)),
        interpret=interpret,
    )(h, t, ri, rel, w, cb, g)


def kernel(x, rel_emb, head_index, rel_type, tail_index, conv_w, conv_b, fc_w):
    B = head_index.shape[0]
    D = x.shape[1]
    out_ch = conv_w.shape[0]

    hi = head_index.astype(jnp.int32)
    ti = tail_index.astype(jnp.int32)
    ri = rel_type.astype(jnp.int32)

    sc_gather = _make_sc_gather(B, D)
    h, t = sc_gather(x, hi, ti)

    w = conv_w.reshape(out_ch, 3)
    g = fc_w.reshape(out_ch, D)
    score, l2s = _tc_score(
        h, t, ri.reshape(B, 1), rel_emb, w, conv_b, g, out_ch=out_ch, d=D)
    return score.reshape(B), l2s[0, 0]
